# Initial kernel scaffold; baseline (speedup 1.0000x reference)
import jax
import jax.numpy as jnp
from jax.experimental import pallas as pl


def kernel(x_seq, h0, W, W_hidden):
    raise NotImplementedError("write your pallas kernel here")


# speedup vs baseline: 1.0377x; 1.0377x over previous
import jax
import jax.numpy as jnp
from jax.experimental import pallas as pl
from jax.experimental.pallas import tpu as pltpu


def _ceil_to(n, m):
    return ((n + m - 1) // m) * m


def _make_body(Tc, HB, n_split):
    def body(x_ref, h0_ref, wt_ref, wh_ref, o_ref, h_ref):
        tc = pl.program_id(1)

        @pl.when(tc == 0)
        def _():
            h_ref[...] = h0_ref[...]

        wh = wh_ref[...]
        wt = wt_ref[...]
        hs = [h_ref[i * HB:(i + 1) * HB] for i in range(n_split)]
        for s in range(Tc):
            xs = x_ref[s].astype(jnp.bfloat16)
            pre = jnp.dot(xs, wt, preferred_element_type=jnp.float32)
            for i in range(n_split):
                rec = jnp.dot(hs[i], wh, preferred_element_type=jnp.float32)
                y = jnp.tanh(pre[i * HB:(i + 1) * HB] + rec)
                o_ref[s, i * HB:(i + 1) * HB] = y
                hs[i] = y.astype(jnp.bfloat16)
        for i in range(n_split):
            h_ref[i * HB:(i + 1) * HB] = hs[i]

    return body


def kernel(x_seq, h0, W, W_hidden):
    T, B, in_dim = x_seq.shape
    out_dim = W.shape[0]
    dtype = x_seq.dtype

    Dp = _ceil_to(in_dim, 128)
    Np = _ceil_to(out_dim, 128)
    TB = 128 if B % 128 == 0 else _ceil_to(B, 8)
    Bp = _ceil_to(B, TB)
    Tc = 8 if T % 8 == 0 else 1
    Tp = _ceil_to(T, Tc)
    n_split = 2 if TB % 16 == 0 else 1
    HB = TB // n_split

    wt = jnp.zeros((Dp, Np), jnp.bfloat16).at[:in_dim, :out_dim].set(
        W.T.astype(jnp.bfloat16))
    wh = jnp.zeros((Dp, Np), jnp.bfloat16).at[:in_dim, :out_dim].set(
        W_hidden.T.astype(jnp.bfloat16))
    h0_p = jnp.zeros((Bp, Dp), jnp.bfloat16).at[:B, :in_dim].set(
        h0.astype(jnp.bfloat16))
    if (Tp, Bp, Dp) != (T, B, in_dim):
        x_p = jnp.zeros((Tp, Bp, Dp), dtype).at[:T, :B, :in_dim].set(x_seq)
    else:
        x_p = x_seq

    cost = pl.CostEstimate(
        flops=2 * 2 * Tp * Bp * Dp * Np,
        transcendentals=Tp * Bp * Np,
        bytes_accessed=4 * (2 * Tp * Bp * Np),
    )

    out_p = pl.pallas_call(
        _make_body(Tc, HB, n_split),
        out_shape=jax.ShapeDtypeStruct((Tp, Bp, Np), jnp.float32),
        grid=(Bp // TB, Tp // Tc),
        in_specs=[
            pl.BlockSpec((Tc, TB, Dp), lambda b, t: (t, b, 0)),
            pl.BlockSpec((TB, Dp), lambda b, t: (b, 0)),
            pl.BlockSpec((Dp, Np), lambda b, t: (0, 0)),
            pl.BlockSpec((Dp, Np), lambda b, t: (0, 0)),
        ],
        out_specs=pl.BlockSpec((Tc, TB, Np), lambda b, t: (t, b, 0)),
        scratch_shapes=[pltpu.VMEM((TB, Dp), jnp.bfloat16)],
        compiler_params=pltpu.CompilerParams(
            dimension_semantics=("parallel", "arbitrary"),
        ),
        cost_estimate=cost,
    )(x_p, h0_p, wt, wh)

    if (Tp, Bp, Np) != (T, B, out_dim):
        out_p = out_p[:T, :B, :out_dim]
    return out_p.astype(dtype)


# speedup vs baseline: 1.7201x; 1.6576x over previous
import jax
import jax.numpy as jnp
from jax.experimental import pallas as pl
from jax.experimental.pallas import tpu as pltpu


def _ceil_to(n, m):
    return ((n + m - 1) // m) * m


def _make_body(Tc, HB, n_split):
    def body(x_ref, h0_ref, wt_ref, wh_ref, o_ref, h_ref):
        tc = pl.program_id(1)

        @pl.when(tc == 0)
        def _():
            h_ref[...] = h0_ref[...]

        wh = wh_ref[...]
        wt = wt_ref[...]
        hs = [h_ref[i * HB:(i + 1) * HB] for i in range(n_split)]
        for s in range(Tc):
            xs = x_ref[s].astype(jnp.bfloat16)
            pre = jnp.dot(xs, wt, preferred_element_type=jnp.float32)
            for i in range(n_split):
                rec = jnp.dot(hs[i], wh, preferred_element_type=jnp.float32)
                y = jnp.tanh(pre[i * HB:(i + 1) * HB] + rec)
                o_ref[s, i * HB:(i + 1) * HB] = y
                hs[i] = y.astype(jnp.bfloat16)
        for i in range(n_split):
            h_ref[i * HB:(i + 1) * HB] = hs[i]

    return body


def kernel(x_seq, h0, W, W_hidden):
    T, B, in_dim = x_seq.shape
    out_dim = W.shape[0]
    dtype = x_seq.dtype

    Dp = _ceil_to(in_dim, 128)
    Np = _ceil_to(out_dim, 128)
    TB = _ceil_to(B, 8)
    Bp = TB
    Tc = 16 if T % 16 == 0 else (8 if T % 8 == 0 else 1)
    Tp = _ceil_to(T, Tc)
    n_split = 2 if TB % 16 == 0 else 1
    HB = TB // n_split

    wt = jnp.zeros((Dp, Np), jnp.bfloat16).at[:in_dim, :out_dim].set(
        W.T.astype(jnp.bfloat16))
    wh = jnp.zeros((Dp, Np), jnp.bfloat16).at[:in_dim, :out_dim].set(
        W_hidden.T.astype(jnp.bfloat16))
    h0_p = jnp.zeros((Bp, Dp), jnp.bfloat16).at[:B, :in_dim].set(
        h0.astype(jnp.bfloat16))
    if (Tp, Bp, Dp) != (T, B, in_dim):
        x_p = jnp.zeros((Tp, Bp, Dp), dtype).at[:T, :B, :in_dim].set(x_seq)
    else:
        x_p = x_seq

    cost = pl.CostEstimate(
        flops=2 * 2 * Tp * Bp * Dp * Np,
        transcendentals=Tp * Bp * Np,
        bytes_accessed=4 * (2 * Tp * Bp * Np),
    )

    out_p = pl.pallas_call(
        _make_body(Tc, HB, n_split),
        out_shape=jax.ShapeDtypeStruct((Tp, Bp, Np), jnp.float32),
        grid=(Bp // TB, Tp // Tc),
        in_specs=[
            pl.BlockSpec((Tc, TB, Dp), lambda b, t: (t, b, 0)),
            pl.BlockSpec((TB, Dp), lambda b, t: (b, 0)),
            pl.BlockSpec((Dp, Np), lambda b, t: (0, 0)),
            pl.BlockSpec((Dp, Np), lambda b, t: (0, 0)),
        ],
        out_specs=pl.BlockSpec((Tc, TB, Np), lambda b, t: (t, b, 0)),
        scratch_shapes=[pltpu.VMEM((TB, Dp), jnp.bfloat16)],
        compiler_params=pltpu.CompilerParams(
            dimension_semantics=("parallel", "arbitrary"),
        ),
        cost_estimate=cost,
    )(x_p, h0_p, wt, wh)

    if (Tp, Bp, Np) != (T, B, out_dim):
        out_p = out_p[:T, :B, :out_dim]
    return out_p.astype(dtype)
